# Initial kernel scaffold; baseline (speedup 1.0000x reference)
import jax
import jax.numpy as jnp
from jax.experimental import pallas as pl


def kernel(lut_count, lut, x):
    raise NotImplementedError("write your pallas kernel here")


# speedup vs baseline: 1677.6606x; 1677.6606x over previous
import functools

import jax
import jax.numpy as jnp
from jax import lax
from jax.experimental import pallas as pl
from jax.experimental.pallas import tpu as pltpu
from jax.experimental.pallas import tpu_sc as plsc

DIM = 33
TBL = DIM * DIM * DIM
NLUT = 3 * TBL
BINSIZE = 1.000001 / (DIM - 1)
INV_BIN = float(1.0 / BINSIZE)


@functools.lru_cache(maxsize=None)
def _build(n_batch, pixels):
  info = plsc.get_sparse_core_info()
  NC, NS, L = info.num_cores, info.num_subcores, info.num_lanes
  NW = NC * NS
  ppw = pixels // NW
  C = 2048
  steps = ppw // C
  chan_stride = pixels
  batch_stride = 3 * pixels

  mesh = plsc.VectorSubcoreMesh(core_axis_name="c", subcore_axis_name="s")

  @functools.partial(
      pl.kernel,
      mesh=mesh,
      compiler_params=pltpu.CompilerParams(needs_layout_passes=False),
      out_type=jax.ShapeDtypeStruct((n_batch * 3 * pixels,), jnp.float32),
      scratch_types=[
          pltpu.VMEM((NLUT,), jnp.float32),
          pltpu.VMEM((C,), jnp.float32),
          pltpu.VMEM((C,), jnp.float32),
          pltpu.VMEM((C,), jnp.float32),
          pltpu.VMEM((C,), jnp.float32),
          pltpu.VMEM((C,), jnp.float32),
          pltpu.VMEM((C,), jnp.float32),
      ],
  )
  def sc_kernel(lut_hbm, x_hbm, out_hbm, lut_v, rv, gv, bv, orv, ogv, obv):
    wid = lax.axis_index("s") * NC + lax.axis_index("c")
    pltpu.sync_copy(lut_hbm, lut_v)
    base0 = wid * ppw

    def step(t, carry):
      b = t // steps
      s = t % steps
      start = b * batch_stride + base0 + s * C
      pltpu.sync_copy(x_hbm.at[pl.ds(start, C)], rv)
      pltpu.sync_copy(x_hbm.at[pl.ds(start + chan_stride, C)], gv)
      pltpu.sync_copy(x_hbm.at[pl.ds(start + 2 * chan_stride, C)], bv)

      def vec(i, c2):
        off = i * L
        rs = rv[pl.ds(off, L)] * INV_BIN
        gs = gv[pl.ds(off, L)] * INV_BIN
        bs = bv[pl.ds(off, L)] * INV_BIN
        ri = rs.astype(jnp.int32)
        gi = gs.astype(jnp.int32)
        bi = bs.astype(jnp.int32)
        rd = rs - ri.astype(jnp.float32)
        gd = gs - gi.astype(jnp.float32)
        bd = bs - bi.astype(jnp.float32)
        rd1 = 1.0 - rd
        gd1 = 1.0 - gd
        bd1 = 1.0 - bd
        w00 = rd1 * gd1
        w10 = rd * gd1
        w01 = rd1 * gd
        w11 = rd * gd
        ws = (w00 * bd1, w10 * bd1, w01 * bd1, w11 * bd1,
              w00 * bd, w10 * bd, w01 * bd, w11 * bd)
        base = ri + gi * DIM + bi * (DIM * DIM)
        offs = (0, 1, DIM, DIM + 1,
                DIM * DIM, DIM * DIM + 1, DIM * DIM + DIM, DIM * DIM + DIM + 1)
        outs = []
        for c in range(3):
          cb = c * TBL
          acc = ws[0] * plsc.load_gather(lut_v, [base + (cb + offs[0])])
          for j in range(1, 8):
            acc = acc + ws[j] * plsc.load_gather(lut_v, [base + (cb + offs[j])])
          outs.append(acc)
        orv[pl.ds(off, L)] = outs[0]
        ogv[pl.ds(off, L)] = outs[1]
        obv[pl.ds(off, L)] = outs[2]
        return c2

      lax.fori_loop(0, C // L, vec, 0)
      pltpu.sync_copy(orv, out_hbm.at[pl.ds(start, C)])
      pltpu.sync_copy(ogv, out_hbm.at[pl.ds(start + chan_stride, C)])
      pltpu.sync_copy(obv, out_hbm.at[pl.ds(start + 2 * chan_stride, C)])
      return carry

    lax.fori_loop(0, n_batch * steps, step, 0)

  return sc_kernel


def kernel(lut_count, lut, x):
  n_batch = x.shape[0]
  pixels = x.shape[2] * x.shape[3]
  fn = _build(n_batch, pixels)
  out = fn(lut.reshape(-1), x.reshape(-1))
  return (lut, out.reshape(x.shape))
